# baseline (speedup 1.0000x reference)
import functools

import jax
import jax.numpy as jnp
from jax import lax
from jax.experimental import pallas as pl
from jax.experimental.pallas import tpu as pltpu
from jax.experimental.pallas import tpu_sc as plsc

B = 16384
D = 32
OUT_D = D + 1

_info = plsc.get_sparse_core_info()
NC, NS, L = _info.num_cores, _info.num_subcores, _info.num_lanes
NW = NC * NS
BPW = B // NW
CHUNK = 128
NCHUNK = BPW // CHUNK

_mesh = plsc.VectorSubcoreMesh(core_axis_name="c", subcore_axis_name="s")


@functools.partial(
    pl.kernel,
    out_type=jax.ShapeDtypeStruct((B * OUT_D,), jnp.float32),
    mesh=_mesh,
    compiler_params=pltpu.CompilerParams(needs_layout_passes=False,
                                         use_tc_tiling_on_sc=False),
    scratch_types=[
        pltpu.VMEM((NCHUNK, CHUNK), jnp.int32),
        pltpu.VMEM((BPW, D), jnp.float32),
        pltpu.VMEM((BPW,), jnp.float32),
        pltpu.VMEM((2, L), jnp.float32),
        pltpu.VMEM((BPW * OUT_D,), jnp.float32),
        pltpu.SemaphoreType.DMA,
    ],
)
def _query_model_sc(uid_hbm, uvt_hbm, par_hbm, table_hbm, out_hbm,
                    idx_v, rows_v, vt_v, par_v, obuf, sem):
    wid = lax.axis_index("s") * NC + lax.axis_index("c")
    base = wid * BPW

    pltpu.sync_copy(uid_hbm.at[wid], idx_v)
    copies = [
        pltpu.async_copy(table_hbm.at[idx_v.at[c]],
                         rows_v.at[pl.ds(c * CHUNK, CHUNK)], sem)
        for c in range(NCHUNK)
    ]
    pltpu.sync_copy(uvt_hbm.at[pl.ds(base, BPW)], vt_v)
    pltpu.sync_copy(par_hbm, par_v)

    scale = par_v[0, :]
    bias = par_v[1, :]

    def norm_grp(j, carry):
        y = vt_v[pl.ds(j * L, L)] * scale + bias
        pos = (j * L + lax.iota(jnp.int32, L)) * OUT_D + D
        plsc.store_scatter(obuf, [pos], y)
        return carry

    lax.fori_loop(0, BPW // L, norm_grp, 0)

    for cp in copies:
        cp.wait()

    def row_copy(i, carry):
        obuf[pl.ds(i * OUT_D, L)] = rows_v[i, pl.ds(0, L)]
        obuf[pl.ds(i * OUT_D + L, L)] = rows_v[i, pl.ds(L, L)]
        return carry

    lax.fori_loop(0, BPW, row_copy, 0)

    pltpu.sync_copy(obuf, out_hbm.at[pl.ds(base * OUT_D, BPW * OUT_D)])


def kernel(user_id, user_view_time, embedding_table, norm_mean, norm_var):
    uid3 = user_id.astype(jnp.int32).reshape(NW, NCHUNK, CHUNK)
    uvt = user_view_time.reshape(B)
    scale = 1.0 / jnp.sqrt(norm_var + 1e-6)
    bias = -norm_mean * scale
    params = jnp.stack([
        jnp.broadcast_to(scale, (L,)),
        jnp.broadcast_to(bias, (L,)),
    ]).astype(jnp.float32)
    flat = _query_model_sc(uid3, uvt, params, embedding_table)
    return flat.reshape(B, OUT_D)
